# baseline (speedup 1.0000x reference)
import functools

import jax
import jax.numpy as jnp
import numpy as np
from jax import lax
from jax.experimental import pallas as pl
from jax.experimental.pallas import tpu as pltpu
from jax.experimental.pallas import tpu_sc as plsc

N_CORE = 16384
N_AUX = 8192
C_IN = 128
C_OUT = 128
KK = 9
RH = 64
RW = 64
NROI = 64
NW = 32
RPW = N_CORE // NW
PAIRS = RPW // 2
IPW = RPW * 4 * KK
CHUNKS_A = IPW // 128
PAD_FLAT = NROI * RH * RW
PAD_ROW = N_CORE + N_AUX
V_EXT = N_CORE + N_AUX + 8
AW = 8
NBUF = 4


def _offsets_tc(x, w_pad, b_full):
    def body(w_ref, x_ref, b_ref, o_ref):
        r = lax.dot_general(w_ref[...], x_ref[...], (((1,), (1,)), ((), ())),
                            preferred_element_type=jnp.float32)
        o_ref[...] = (r + b_ref[...])[None]

    return pl.pallas_call(
        body,
        grid=(NW,),
        in_specs=[
            pl.BlockSpec((24, C_IN), lambda i: (0, 0)),
            pl.BlockSpec((RPW, C_IN), lambda i: (i, 0)),
            pl.BlockSpec((24, RPW), lambda i: (0, 0)),
        ],
        out_specs=pl.BlockSpec((1, 24, RPW), lambda i: (i, 0, 0)),
        out_shape=jax.ShapeDtypeStruct((NW, 24, RPW), jnp.float32),
    )(w_pad, x, b_full)


def _matmul_tc(agg, weight, bias8):
    def body(a_ref, w_ref, b_ref, o_ref):
        r = lax.dot_general(a_ref[...], w_ref[...], (((1,), (1,)), ((), ())),
                            preferred_element_type=jnp.float32)
        o_ref[...] = r + b_ref[0:1, :]

    return pl.pallas_call(
        body,
        grid=(64,),
        in_specs=[
            pl.BlockSpec((256, KK * C_IN), lambda i: (i, 0)),
            pl.BlockSpec((C_OUT, KK * C_IN), lambda i: (0, 0)),
            pl.BlockSpec((8, C_OUT), lambda i: (0, 0)),
        ],
        out_specs=pl.BlockSpec((256, C_OUT), lambda i: (i, 0)),
        out_shape=jax.ShapeDtypeStruct((N_CORE, C_OUT), jnp.float32),
    )(agg, weight, bias8)


_CORNERS = ((0, 0), (1, 0), (0, 1), (1, 1))


def _sc_body(cxy_h, pos_h, roi_h, idm_h, feats_h, agg_h,
             cxy_v, pos_v, roi_v, imidx_v, wgt_v, ids_v, rbuf, obuf,
             semA, gsem, osem0, osem1, osem2, osem3):
    wid = lax.axis_index("s") * 2 + lax.axis_index("c")
    base = wid * RPW

    pltpu.sync_copy(cxy_h.at[wid], cxy_v)
    pltpu.sync_copy(pos_h.at[wid], pos_v)
    pltpu.sync_copy(roi_h.at[wid], roi_v)

    lanes = lax.iota(jnp.int32, 16)

    @pl.loop(0, RPW // 16)
    def _phase0(g):
        n0 = g * 16
        px = pos_v[0, pl.ds(n0, 16)].astype(jnp.float32)
        py = pos_v[1, pl.ds(n0, 16)].astype(jnp.float32)
        rbase = roi_v[pl.ds(n0, 16)] * (RH * RW)
        sbase = (lanes + n0) * (4 * KK)
        for k in range(KK):
            x = cxy_v[2 * k, pl.ds(n0, 16)] + px
            y = cxy_v[2 * k + 1, pl.ds(n0, 16)] + py
            fx = x.astype(jnp.int32)
            fx = jnp.where(x < fx.astype(jnp.float32), fx - 1, fx)
            fy = y.astype(jnp.int32)
            fy = jnp.where(y < fy.astype(jnp.float32), fy - 1, fy)
            dx = x - fx.astype(jnp.float32)
            dy = y - fy.astype(jnp.float32)
            for c, (gx, gy) in enumerate(_CORNERS):
                cx = fx + gx
                cy = fy + gy
                padm = (cx < 0) | (cy < 0) | (cx >= RW) | (cy >= RH)
                cxc = jnp.clip(cx, 0, RW - 1)
                cyc = jnp.clip(cy, 0, RH - 1)
                flat = jnp.where(padm, PAD_FLAT, rbase + cyc * RW + cxc)
                f1 = (dy, dx, dy, dx)[c]
                f2 = (1.0 - dy, 1.0 - dy, 1.0 - dx, 1.0 - dx)[c]
                slot = sbase + (4 * k + c)
                plsc.store_scatter(imidx_v, [slot], flat)
                plsc.store_scatter(wgt_v, [slot], f1 * f2)

    @pl.loop(0, CHUNKS_A)
    def _phaseA(t):
        pltpu.async_copy(idm_h.at[imidx_v.at[pl.ds(t * 128, 128)]],
                         ids_v.at[pl.ds(t * 128, 128)], semA)

        @pl.when(t >= AW)
        def _():
            pltpu.make_async_copy(idm_h.at[pl.ds(0, 128)],
                                  ids_v.at[pl.ds(0, 128)], semA).wait()

    @pl.loop(0, AW)
    def _drainA(t):
        pltpu.make_async_copy(idm_h.at[pl.ds(0, 128)],
                              ids_v.at[pl.ds(0, 128)], semA).wait()

    osems = (osem0, osem1, osem2, osem3)
    for b in range(NBUF):
        pltpu.async_copy(feats_h.at[ids_v.at[pl.ds(b * 72, 72)]],
                         rbuf.at[b], gsem)

    @pl.loop(0, PAIRS // NBUF)
    def _phaseB(g):
        for b in range(NBUF):
            p = g * NBUF + b
            pltpu.make_async_copy(feats_h.at[pl.ds(0, 72)], rbuf.at[b],
                                  gsem).wait()

            @pl.when(g > 0)
            def _():
                pltpu.make_async_copy(obuf.at[b], agg_h.at[pl.ds(0, 2)],
                                      osems[b]).wait()

            wbase = p * 72
            for r in range(2):
                for k in range(KK):
                    acc = [None] * 8
                    for c in range(4):
                        j = r * 36 + 4 * k + c
                        wv = plsc.load_gather(
                            wgt_v, [jnp.full((16,), wbase + j, jnp.int32)])
                        for q in range(4):
                            ab = plsc.bitcast(rbuf[b, j, pl.ds(q * 16, 16)],
                                              jnp.bfloat16)
                            ev, od = plsc.unpack(
                                ab, format=plsc.PackFormat.INTERLEAVED)
                            tev = wv * ev
                            tod = wv * od
                            acc[2 * q] = tev if c == 0 else acc[2 * q] + tev
                            acc[2 * q + 1] = (tod if c == 0
                                              else acc[2 * q + 1] + tod)
                    for q in range(4):
                        pk = plsc.pack(acc[2 * q], acc[2 * q + 1],
                                       format=plsc.PackFormat.INTERLEAVED)
                        obuf[b, r, pl.ds(k * 64 + q * 16, 16)] = plsc.bitcast(
                            pk, jnp.int32)

            pltpu.async_copy(obuf.at[b], agg_h.at[pl.ds(base + p * 2, 2)],
                             osems[b])

            @pl.when(p + NBUF < PAIRS)
            def _():
                pltpu.async_copy(
                    feats_h.at[ids_v.at[pl.ds((p + NBUF) * 72, 72)]],
                    rbuf.at[b], gsem)

    for b in range(NBUF):
        pltpu.make_async_copy(obuf.at[b], agg_h.at[pl.ds(0, 2)],
                              osems[b]).wait()


@functools.partial(jax.jit, static_argnums=())
def _sc_main(cxy_w, pos_w, roi_w, id_ext, feats):
    mesh = plsc.VectorSubcoreMesh(core_axis_name="c", subcore_axis_name="s")
    return pl.kernel(
        _sc_body,
        out_type=jax.ShapeDtypeStruct((N_CORE, KK * 64), jnp.int32),
        mesh=mesh,
        compiler_params=pltpu.CompilerParams(needs_layout_passes=False, use_tc_tiling_on_sc=False),
        scratch_types=[
            pltpu.VMEM((24, RPW), jnp.float32),
            pltpu.VMEM((2, RPW), jnp.int32),
            pltpu.VMEM((RPW,), jnp.int32),
            pltpu.VMEM((IPW,), jnp.int32),
            pltpu.VMEM((IPW,), jnp.float32),
            pltpu.VMEM((IPW,), jnp.int32),
            pltpu.VMEM((NBUF, 72, C_IN // 2), jnp.int32),
            pltpu.VMEM((NBUF, 2, KK * 64), jnp.int32),
            pltpu.SemaphoreType.DMA,
            pltpu.SemaphoreType.DMA,
            pltpu.SemaphoreType.DMA,
            pltpu.SemaphoreType.DMA,
            pltpu.SemaphoreType.DMA,
            pltpu.SemaphoreType.DMA,
        ],
    )(cxy_w, pos_w, roi_w, id_ext, feats)


def kernel(in_core_feats, aux_feats, id_map, roi_ids, pos_ids, W_off, b_off,
           weight, bias):
    feats_bf = jnp.concatenate(
        [in_core_feats, aux_feats, jnp.zeros((8, C_IN), jnp.float32)],
        axis=0).astype(jnp.bfloat16)
    feats = lax.bitcast_convert_type(
        feats_bf.reshape(V_EXT, C_IN // 2, 2), jnp.int32)
    id_ext = jnp.concatenate(
        [id_map.reshape(-1).astype(jnp.int32),
         jnp.full((8,), PAD_ROW, jnp.int32)])
    roi_w = roi_ids.astype(jnp.int32).reshape(NW, RPW)
    pos_w = pos_ids.astype(jnp.int32).T.reshape(2, NW, RPW).transpose(1, 0, 2)
    W_off_pad = jnp.zeros((24, C_IN), jnp.float32).at[:18].set(W_off)
    boff24 = jnp.zeros((24,), jnp.float32).at[:18].set(b_off)
    b_full = jnp.broadcast_to(boff24.reshape(24, 1), (24, RPW))
    bias8 = jnp.broadcast_to(bias.reshape(1, C_OUT), (8, C_OUT))

    cxy_w = _offsets_tc(in_core_feats, W_off_pad, b_full)
    agg_i = _sc_main(cxy_w, pos_w, roi_w, id_ext, feats)
    agg = lax.bitcast_convert_type(agg_i, jnp.bfloat16).reshape(
        N_CORE, KK * C_IN)
    return _matmul_tc(agg, weight.astype(jnp.bfloat16), bias8)


# speedup vs baseline: 1.3679x; 1.3679x over previous
import functools

import jax
import jax.numpy as jnp
import numpy as np
from jax import lax
from jax.experimental import pallas as pl
from jax.experimental.pallas import tpu as pltpu
from jax.experimental.pallas import tpu_sc as plsc

N_CORE = 16384
N_AUX = 8192
C_IN = 128
C_OUT = 128
KK = 9
RH = 64
RW = 64
NROI = 64
NW = 32
RPW = N_CORE // NW
PAIRS = RPW // 2
IPW = RPW * 4 * KK
CHUNKS_A = IPW // 128
PAD_FLAT = NROI * RH * RW
PAD_ROW = N_CORE + N_AUX
V_EXT = N_CORE + N_AUX + 8
AW = 8
NBUF = 4


def _offsets_tc(x, w_pad, b_full):
    def body(w_ref, x_ref, b_ref, o_ref):
        r = lax.dot_general(w_ref[...], x_ref[...], (((1,), (1,)), ((), ())),
                            preferred_element_type=jnp.float32)
        o_ref[...] = (r + b_ref[...])[None]

    return pl.pallas_call(
        body,
        grid=(NW,),
        in_specs=[
            pl.BlockSpec((24, C_IN), lambda i: (0, 0)),
            pl.BlockSpec((RPW, C_IN), lambda i: (i, 0)),
            pl.BlockSpec((24, RPW), lambda i: (0, 0)),
        ],
        out_specs=pl.BlockSpec((1, 24, RPW), lambda i: (i, 0, 0)),
        out_shape=jax.ShapeDtypeStruct((NW, 24, RPW), jnp.float32),
    )(w_pad, x, b_full)


def _matmul_tc(agg, weight, bias8):
    def body(a_ref, w_ref, b_ref, o_ref):
        a = a_ref[...]
        lo = lax.bitcast_convert_type(
            (a & 0xFFFF).astype(jnp.uint16), jnp.bfloat16)
        hi = lax.bitcast_convert_type(
            lax.shift_right_logical(a, 16).astype(jnp.uint16), jnp.bfloat16)
        abf = jnp.concatenate([lo, hi], axis=1)
        r = lax.dot_general(abf, w_ref[...], (((1,), (1,)), ((), ())),
                            preferred_element_type=jnp.float32)
        o_ref[...] = r + b_ref[0:1, :]

    return pl.pallas_call(
        body,
        grid=(64,),
        in_specs=[
            pl.BlockSpec((256, KK * 64), lambda i: (i, 0)),
            pl.BlockSpec((C_OUT, KK * C_IN), lambda i: (0, 0)),
            pl.BlockSpec((8, C_OUT), lambda i: (0, 0)),
        ],
        out_specs=pl.BlockSpec((256, C_OUT), lambda i: (i, 0)),
        out_shape=jax.ShapeDtypeStruct((N_CORE, C_OUT), jnp.float32),
    )(agg, weight, bias8)


_CORNERS = ((0, 0), (1, 0), (0, 1), (1, 1))

_W = np.arange(KK * 64)
_LO = (_W // 64) * 128 + 32 * ((_W % 64) // 16) + 2 * (_W % 16)
_WPERM = np.concatenate([_LO, _LO + 1])


def _sc_body(cxy_h, pos_h, roi_h, idm_h, feats_h, agg_h,
             cxy_v, pos_v, roi_v, imidx_v, wgt_v, ids_v, rbuf, obuf,
             semA, gsem, osem0, osem1, osem2, osem3):
    wid = lax.axis_index("s") * 2 + lax.axis_index("c")
    base = wid * RPW

    pltpu.sync_copy(cxy_h.at[wid], cxy_v)
    pltpu.sync_copy(pos_h.at[wid], pos_v)
    pltpu.sync_copy(roi_h.at[wid], roi_v)

    lanes = lax.iota(jnp.int32, 16)

    @pl.loop(0, RPW // 16)
    def _phase0(g):
        n0 = g * 16
        px = pos_v[0, pl.ds(n0, 16)].astype(jnp.float32)
        py = pos_v[1, pl.ds(n0, 16)].astype(jnp.float32)
        rbase = roi_v[pl.ds(n0, 16)] * (RH * RW)
        sbase = (lanes + n0) * (4 * KK)
        for k in range(KK):
            x = cxy_v[2 * k, pl.ds(n0, 16)] + px
            y = cxy_v[2 * k + 1, pl.ds(n0, 16)] + py
            fx = x.astype(jnp.int32)
            fx = jnp.where(x < fx.astype(jnp.float32), fx - 1, fx)
            fy = y.astype(jnp.int32)
            fy = jnp.where(y < fy.astype(jnp.float32), fy - 1, fy)
            dx = x - fx.astype(jnp.float32)
            dy = y - fy.astype(jnp.float32)
            for c, (gx, gy) in enumerate(_CORNERS):
                cx = fx + gx
                cy = fy + gy
                padm = (cx < 0) | (cy < 0) | (cx >= RW) | (cy >= RH)
                cxc = jnp.clip(cx, 0, RW - 1)
                cyc = jnp.clip(cy, 0, RH - 1)
                flat = jnp.where(padm, PAD_FLAT, rbase + cyc * RW + cxc)
                f1 = (dy, dx, dy, dx)[c]
                f2 = (1.0 - dy, 1.0 - dy, 1.0 - dx, 1.0 - dx)[c]
                slot = sbase + (4 * k + c)
                plsc.store_scatter(imidx_v, [slot], flat)
                plsc.store_scatter(wgt_v, [slot], f1 * f2)

    @pl.loop(0, CHUNKS_A)
    def _phaseA(t):
        pltpu.async_copy(idm_h.at[imidx_v.at[pl.ds(t * 128, 128)]],
                         ids_v.at[pl.ds(t * 128, 128)], semA)

        @pl.when(t >= AW)
        def _():
            pltpu.make_async_copy(idm_h.at[pl.ds(0, 128)],
                                  ids_v.at[pl.ds(0, 128)], semA).wait()

    @pl.loop(0, AW)
    def _drainA(t):
        pltpu.make_async_copy(idm_h.at[pl.ds(0, 128)],
                              ids_v.at[pl.ds(0, 128)], semA).wait()

    osems = (osem0, osem1, osem2, osem3)
    for b in range(NBUF):
        pltpu.async_copy(feats_h.at[ids_v.at[pl.ds(b * 72, 72)]],
                         rbuf.at[b], gsem)

    @pl.loop(0, PAIRS // NBUF)
    def _phaseB(g):
        for b in range(NBUF):
            p = g * NBUF + b
            pltpu.make_async_copy(feats_h.at[pl.ds(0, 72)], rbuf.at[b],
                                  gsem).wait()

            @pl.when(g > 0)
            def _():
                pltpu.make_async_copy(obuf.at[b], agg_h.at[pl.ds(0, 2)],
                                      osems[b]).wait()

            wbase = p * 72
            for r in range(2):
                for k in range(KK):
                    acc = [None] * 8
                    for c in range(4):
                        j = r * 36 + 4 * k + c
                        wv = plsc.load_gather(
                            wgt_v, [jnp.full((16,), wbase + j, jnp.int32)])
                        for q in range(4):
                            ab = plsc.bitcast(rbuf[b, j, pl.ds(q * 16, 16)],
                                              jnp.bfloat16)
                            ev, od = plsc.unpack(
                                ab, format=plsc.PackFormat.INTERLEAVED)
                            tev = wv * ev
                            tod = wv * od
                            acc[2 * q] = tev if c == 0 else acc[2 * q] + tev
                            acc[2 * q + 1] = (tod if c == 0
                                              else acc[2 * q + 1] + tod)
                    for q in range(4):
                        pk = plsc.pack(acc[2 * q], acc[2 * q + 1],
                                       format=plsc.PackFormat.INTERLEAVED)
                        obuf[b, r, pl.ds(k * 64 + q * 16, 16)] = plsc.bitcast(
                            pk, jnp.int32)

            pltpu.async_copy(obuf.at[b], agg_h.at[pl.ds(base + p * 2, 2)],
                             osems[b])

            @pl.when(p + NBUF < PAIRS)
            def _():
                pltpu.async_copy(
                    feats_h.at[ids_v.at[pl.ds((p + NBUF) * 72, 72)]],
                    rbuf.at[b], gsem)

    for b in range(NBUF):
        pltpu.make_async_copy(obuf.at[b], agg_h.at[pl.ds(0, 2)],
                              osems[b]).wait()


@functools.partial(jax.jit, static_argnums=())
def _sc_main(cxy_w, pos_w, roi_w, id_ext, feats):
    mesh = plsc.VectorSubcoreMesh(core_axis_name="c", subcore_axis_name="s")
    return pl.kernel(
        _sc_body,
        out_type=jax.ShapeDtypeStruct((N_CORE, KK * 64), jnp.int32),
        mesh=mesh,
        compiler_params=pltpu.CompilerParams(needs_layout_passes=False, use_tc_tiling_on_sc=False),
        scratch_types=[
            pltpu.VMEM((24, RPW), jnp.float32),
            pltpu.VMEM((2, RPW), jnp.int32),
            pltpu.VMEM((RPW,), jnp.int32),
            pltpu.VMEM((IPW,), jnp.int32),
            pltpu.VMEM((IPW,), jnp.float32),
            pltpu.VMEM((IPW,), jnp.int32),
            pltpu.VMEM((NBUF, 72, C_IN // 2), jnp.int32),
            pltpu.VMEM((NBUF, 2, KK * 64), jnp.int32),
            pltpu.SemaphoreType.DMA,
            pltpu.SemaphoreType.DMA,
            pltpu.SemaphoreType.DMA,
            pltpu.SemaphoreType.DMA,
            pltpu.SemaphoreType.DMA,
            pltpu.SemaphoreType.DMA,
        ],
    )(cxy_w, pos_w, roi_w, id_ext, feats)


def kernel(in_core_feats, aux_feats, id_map, roi_ids, pos_ids, W_off, b_off,
           weight, bias):
    feats_bf = jnp.concatenate(
        [in_core_feats, aux_feats, jnp.zeros((8, C_IN), jnp.float32)],
        axis=0).astype(jnp.bfloat16)
    feats = lax.bitcast_convert_type(
        feats_bf.reshape(V_EXT, C_IN // 2, 2), jnp.int32)
    id_ext = jnp.concatenate(
        [id_map.reshape(-1).astype(jnp.int32),
         jnp.full((8,), PAD_ROW, jnp.int32)])
    roi_w = roi_ids.astype(jnp.int32).reshape(NW, RPW)
    pos_w = pos_ids.astype(jnp.int32).T.reshape(2, NW, RPW).transpose(1, 0, 2)
    W_off_pad = jnp.zeros((24, C_IN), jnp.float32).at[:18].set(W_off)
    boff24 = jnp.zeros((24,), jnp.float32).at[:18].set(b_off)
    b_full = jnp.broadcast_to(boff24.reshape(24, 1), (24, RPW))
    bias8 = jnp.broadcast_to(bias.reshape(1, C_OUT), (8, C_OUT))

    cxy_w = _offsets_tc(in_core_feats, W_off_pad, b_full)
    agg_i = _sc_main(cxy_w, pos_w, roi_w, id_ext, feats)
    return _matmul_tc(agg_i, weight[:, _WPERM].astype(jnp.bfloat16), bias8)
